# baseline (speedup 1.0000x reference)
import jax
import jax.numpy as jnp
from jax import lax
from jax.experimental import pallas as pl
from jax.experimental.pallas import tpu as pltpu
from jax.experimental.pallas import tpu_sc as plsc

N = 10000
E = 320000
D = 128

NC = 2
NS = 16
NW = NC * NS
CH = 80
CH2 = CH // 2
EPAD = NW * CH * 128
TBL = 10240
STRIPE = TBL // NS
KS = STRIPE // 128
DUMMY = N

BM = 2000
G = N // BM

_mesh = plsc.VectorSubcoreMesh(core_axis_name="c", subcore_axis_name="s")


def _deg_body(dst3, ones128, zeros128, out, dstv, onesv, table, ssem):
    c = lax.axis_index("c")
    s = lax.axis_index("s")
    wid = c * NS + s
    base = s * STRIPE
    for k in range(KS):
        pltpu.sync_copy(zeros128, table.at[pl.ds(base + k * 128, 128)])
    pltpu.sync_copy(dst3.at[wid], dstv)
    pltpu.sync_copy(ones128, onesv)
    plsc.subcore_barrier()

    def body(t, carry):
        for b in range(4):
            pltpu.async_copy(onesv, table.at[dstv.at[t * 4 + b]], ssem,
                             add=True)
        for b in range(4):
            pltpu.make_async_copy(onesv, table.at[dstv.at[t * 4 + b]],
                                  ssem).wait()
        return carry

    lax.fori_loop(0, CH // 4, body, 0)
    plsc.subcore_barrier()
    for k in range(KS):
        pltpu.sync_copy(table.at[pl.ds(base + k * 128, 128)],
                        out.at[c, pl.ds(base + k * 128, 128)])


_deg = pl.kernel(
    _deg_body,
    mesh=_mesh,
    out_type=jax.ShapeDtypeStruct((NC, TBL, D), jnp.float32),
    scratch_types=[
        pltpu.VMEM((CH, 128), jnp.int32),
        pltpu.VMEM((128, D), jnp.float32),
        pltpu.VMEM_SHARED((TBL, D), jnp.float32),
        pltpu.SemaphoreType.DMA,
    ],
)


def _layer_body(mp, src3, dst3, zeros128, out, srcv, dstv, rowsA, rowsB,
                table, gsemA, gsemB, ssemA, ssemB):
    c = lax.axis_index("c")
    s = lax.axis_index("s")
    wid = c * NS + s
    base = s * STRIPE
    for k in range(KS):
        pltpu.sync_copy(zeros128, table.at[pl.ds(base + k * 128, 128)])
    def fire_gather(j, rows, gsem):
        pltpu.async_copy(mp.at[srcv.at[j]], rows, gsem)

    def drain_gather(j, rows, gsem):
        pltpu.make_async_copy(mp.at[srcv.at[j]], rows, gsem).wait()

    def fire_scatter(j, rows, ssem):
        pltpu.async_copy(rows, table.at[dstv.at[j]], ssem, add=True)

    def drain_scatter(j, rows, ssem):
        pltpu.make_async_copy(rows, table.at[dstv.at[j]], ssem).wait()

    for h in range(2):
        pltpu.sync_copy(src3.at[wid, pl.ds(h * CH2, CH2)], srcv)
        pltpu.sync_copy(dst3.at[wid, pl.ds(h * CH2, CH2)], dstv)
        if h == 0:
            plsc.subcore_barrier()
        fire_gather(0, rowsA, gsemA)

        def body(t, carry):
            jA = 2 * t
            jB = 2 * t + 1
            drain_gather(jA, rowsA, gsemA)
            fire_gather(jB, rowsB, gsemB)
            fire_scatter(jA, rowsA, ssemA)
            drain_scatter(jA, rowsA, ssemA)
            drain_gather(jB, rowsB, gsemB)

            @pl.when(t < CH2 // 2 - 1)
            def _():
                fire_gather(jA + 2, rowsA, gsemA)

            fire_scatter(jB, rowsB, ssemB)
            drain_scatter(jB, rowsB, ssemB)
            return carry

        lax.fori_loop(0, CH2 // 2, body, 0)
    plsc.subcore_barrier()
    for k in range(KS):
        pltpu.sync_copy(table.at[pl.ds(base + k * 128, 128)],
                        out.at[c, pl.ds(base + k * 128, 128)])


_layer = pl.kernel(
    _layer_body,
    mesh=_mesh,
    out_type=jax.ShapeDtypeStruct((NC, TBL, D), jnp.float32),
    scratch_types=[
        pltpu.VMEM((CH2, 128), jnp.int32),
        pltpu.VMEM((CH2, 128), jnp.int32),
        pltpu.VMEM((128, D), jnp.float32),
        pltpu.VMEM((128, D), jnp.float32),
        pltpu.VMEM_SHARED((TBL, D), jnp.float32),
        pltpu.SemaphoreType.DMA,
        pltpu.SemaphoreType.DMA,
        pltpu.SemaphoreType.DMA,
        pltpu.SemaphoreType.DMA,
    ],
)


def _enc_body(x_ref, we_ref, be_ref, wg_ref, out_ref):
    h = jnp.maximum(
        jnp.dot(x_ref[...], we_ref[...], preferred_element_type=jnp.float32)
        + be_ref[...], 0.0)
    out_ref[...] = jnp.dot(h, wg_ref[...], preferred_element_type=jnp.float32)


_enc = pl.pallas_call(
    _enc_body,
    grid=(G,),
    in_specs=[
        pl.BlockSpec((BM, D), lambda i: (i, 0)),
        pl.BlockSpec((D, D), lambda i: (0, 0)),
        pl.BlockSpec((1, D), lambda i: (0, 0)),
        pl.BlockSpec((D, D), lambda i: (0, 0)),
    ],
    out_specs=pl.BlockSpec((BM, D), lambda i: (i, 0)),
    out_shape=jax.ShapeDtypeStruct((N, D), jnp.float32),
)


def _scale_body(hm_ref, cnt_ref, mp_ref, dis_ref):
    deg = 1.0 + cnt_ref[0, :, 0:1] + cnt_ref[1, :, 0:1]
    dis = lax.rsqrt(deg)
    dis_ref[...] = jnp.broadcast_to(dis, (BM, D))
    mp_ref[...] = hm_ref[...] * dis


_scale = pl.pallas_call(
    _scale_body,
    grid=(G,),
    in_specs=[
        pl.BlockSpec((BM, D), lambda i: (i, 0)),
        pl.BlockSpec((NC, BM, D), lambda i: (0, i, 0)),
    ],
    out_specs=[
        pl.BlockSpec((BM, D), lambda i: (i, 0)),
        pl.BlockSpec((BM, D), lambda i: (i, 0)),
    ],
    out_shape=[
        jax.ShapeDtypeStruct((N, D), jnp.float32),
        jax.ShapeDtypeStruct((N, D), jnp.float32),
    ],
)


def _combine_body(parts_ref, mp_ref, dis_ref, b_ref, w_ref, out_ref):
    dis = dis_ref[...]
    q = parts_ref[0] + parts_ref[1] + mp_ref[...]
    h = jnp.maximum(dis * q + b_ref[...], 0.0)
    out_ref[...] = jnp.dot(
        h, w_ref[...], preferred_element_type=jnp.float32) * dis


_combine = pl.pallas_call(
    _combine_body,
    grid=(G,),
    in_specs=[
        pl.BlockSpec((NC, BM, D), lambda i: (0, i, 0)),
        pl.BlockSpec((BM, D), lambda i: (i, 0)),
        pl.BlockSpec((BM, D), lambda i: (i, 0)),
        pl.BlockSpec((1, D), lambda i: (0, 0)),
        pl.BlockSpec((D, D), lambda i: (0, 0)),
    ],
    out_specs=pl.BlockSpec((BM, D), lambda i: (i, 0)),
    out_shape=jax.ShapeDtypeStruct((N, D), jnp.float32),
)


def _final_body(parts_ref, mp_ref, dis_ref, b_ref, wp1_ref, bp1_ref,
                wp2_ref, bp2_ref, out_ref, acc_ref):
    i = pl.program_id(0)

    @pl.when(i == 0)
    def _():
        acc_ref[...] = jnp.zeros_like(acc_ref)

    h = jnp.maximum(
        dis_ref[...] * (parts_ref[0] + parts_ref[1] + mp_ref[...])
        + b_ref[...], 0.0)
    acc_ref[...] += jnp.sum(h, axis=0, keepdims=True)

    @pl.when(i == G - 1)
    def _():
        g = acc_ref[...] * (1.0 / N)
        z = jnp.maximum(
            jnp.dot(g, wp1_ref[...], preferred_element_type=jnp.float32)
            + bp1_ref[...], 0.0)
        out_ref[...] = jnp.dot(
            z, wp2_ref[...], preferred_element_type=jnp.float32) + bp2_ref[...]


_final = pl.pallas_call(
    _final_body,
    grid=(G,),
    in_specs=[
        pl.BlockSpec((NC, BM, D), lambda i: (0, i, 0)),
        pl.BlockSpec((BM, D), lambda i: (i, 0)),
        pl.BlockSpec((BM, D), lambda i: (i, 0)),
        pl.BlockSpec((1, D), lambda i: (0, 0)),
        pl.BlockSpec((D, D // 2), lambda i: (0, 0)),
        pl.BlockSpec((1, D // 2), lambda i: (0, 0)),
        pl.BlockSpec((D // 2, D), lambda i: (0, 0)),
        pl.BlockSpec((1, D), lambda i: (0, 0)),
    ],
    out_specs=pl.BlockSpec((1, D), lambda i: (0, 0)),
    out_shape=jax.ShapeDtypeStruct((1, D), jnp.float32),
    scratch_shapes=[pltpu.VMEM((1, D), jnp.float32)],
)


def kernel(x, edge_index, W_enc, b_enc, W_g0, b_g0, W_g1, b_g1, W_g2, b_g2,
           W_p1, b_p1, W_p2, b_p2):
    src = edge_index[0]
    dst = edge_index[1]
    pad = EPAD - E
    src3 = jnp.concatenate(
        [src, jnp.zeros((pad,), jnp.int32)]).reshape(NW, CH, 128)
    pad_dst = DUMMY + jnp.arange(pad, dtype=jnp.int32) % (TBL - N)
    dst3 = jnp.concatenate([dst, pad_dst]).reshape(NW, CH, 128)
    ones128 = jnp.ones((128, D), jnp.float32)
    zeros128 = jnp.zeros((128, D), jnp.float32)

    counts = _deg(dst3, ones128, zeros128)
    h0m = _enc(x, W_enc, b_enc.reshape(1, D), W_g0)
    mp, disb = _scale(h0m, counts)
    parts = _layer(mp, src3, dst3, zeros128)
    mp = _combine(parts, mp, disb, b_g0.reshape(1, D), W_g1)
    parts = _layer(mp, src3, dst3, zeros128)
    mp = _combine(parts, mp, disb, b_g1.reshape(1, D), W_g2)
    parts = _layer(mp, src3, dst3, zeros128)
    out = _final(parts, mp, disb, b_g2.reshape(1, D), W_p1,
                 b_p1.reshape(1, D // 2), W_p2, b_p2.reshape(1, D))
    return out.reshape(D)


# speedup vs baseline: 1.1158x; 1.1158x over previous
import jax
import jax.numpy as jnp
from jax import lax
from jax.experimental import pallas as pl
from jax.experimental.pallas import tpu as pltpu
from jax.experimental.pallas import tpu_sc as plsc

N = 10000
E = 320000
D = 128

NC = 2
NS = 16
NW = NC * NS
CH = 80
C0 = 128
C1 = 32
GS = 32
TCH = NS * (C0 + C1)
EPAD = TCH * 128
TBL = 10240
STRIPE = TBL // NS
KS = STRIPE // 128
DUMMY = N

BM = 2000
G = N // BM

_mesh = plsc.VectorSubcoreMesh(core_axis_name="c", subcore_axis_name="s")


def _deg_body(dst2, ones128, zeros128, out, dstv, onesv, table, ssem):
    c = lax.axis_index("c")
    s = lax.axis_index("s")
    wid = c * NS + s
    base = s * STRIPE
    for k in range(KS):
        pltpu.sync_copy(zeros128, table.at[pl.ds(base + k * 128, 128)])
    pltpu.sync_copy(dst2.at[pl.ds(wid * CH, CH)], dstv)
    pltpu.sync_copy(ones128, onesv)
    plsc.subcore_barrier()

    def body(t, carry):
        for b in range(4):
            pltpu.async_copy(onesv, table.at[dstv.at[t * 4 + b]], ssem,
                             add=True)
        for b in range(4):
            pltpu.make_async_copy(onesv, table.at[dstv.at[t * 4 + b]],
                                  ssem).wait()
        return carry

    lax.fori_loop(0, CH // 4, body, 0)
    plsc.subcore_barrier()
    for k in range(KS):
        pltpu.sync_copy(table.at[pl.ds(base + k * 128, 128)],
                        out.at[c, pl.ds(base + k * 128, 128)])


_deg = pl.kernel(
    _deg_body,
    mesh=_mesh,
    out_type=jax.ShapeDtypeStruct((NC, TBL, D), jnp.float32),
    scratch_types=[
        pltpu.VMEM((CH, 128), jnp.int32),
        pltpu.VMEM((128, D), jnp.float32),
        pltpu.VMEM_SHARED((TBL, D), jnp.float32),
        pltpu.SemaphoreType.DMA,
    ],
)


def _layer_body(mp, src3, dst3, zeros128, out, srcv, dstv, rowsA, rowsB,
                table, gsemA, gsemB, ssemA, ssemB):
    c = lax.axis_index("c")
    s = lax.axis_index("s")
    wid = c * NS + s
    base = s * STRIPE
    for k in range(KS):
        pltpu.sync_copy(zeros128, table.at[pl.ds(base + k * 128, 128)])
    def fire_gather(j, rows, gsem):
        pltpu.async_copy(mp.at[srcv.at[j]], rows, gsem)

    def drain_gather(j, rows, gsem):
        pltpu.make_async_copy(mp.at[srcv.at[j]], rows, gsem).wait()

    def fire_scatter(j, rows, ssem):
        pltpu.async_copy(rows, table.at[dstv.at[j]], ssem, add=True)

    def drain_scatter(j, rows, ssem):
        pltpu.make_async_copy(rows, table.at[dstv.at[j]], ssem).wait()

    plsc.subcore_barrier()

    cb = lax.select(c == 0, s * C0, NS * C0 + s * C1)
    for g in range(C0 // GS):
        @pl.when((c == 0) | (g == 0))
        def _():
            off = pl.multiple_of(cb + g * GS, 8)
            pltpu.sync_copy(src3.at[pl.ds(off, GS)], srcv)
            pltpu.sync_copy(dst3.at[pl.ds(off, GS)], dstv)
            fire_gather(0, rowsA, gsemA)

            def body(t, carry):
                jA = 2 * t
                jB = 2 * t + 1
                drain_gather(jA, rowsA, gsemA)
                fire_gather(jB, rowsB, gsemB)
                fire_scatter(jA, rowsA, ssemA)
                drain_scatter(jA, rowsA, ssemA)
                drain_gather(jB, rowsB, gsemB)

                @pl.when(t < GS // 2 - 1)
                def _():
                    fire_gather(jA + 2, rowsA, gsemA)

                fire_scatter(jB, rowsB, ssemB)
                drain_scatter(jB, rowsB, ssemB)
                return carry

            lax.fori_loop(0, GS // 2, body, 0)
    plsc.subcore_barrier()
    for k in range(KS):
        pltpu.sync_copy(table.at[pl.ds(base + k * 128, 128)],
                        out.at[c, pl.ds(base + k * 128, 128)])


_layer = pl.kernel(
    _layer_body,
    mesh=_mesh,
    out_type=jax.ShapeDtypeStruct((NC, TBL, D), jnp.float32),
    scratch_types=[
        pltpu.VMEM((GS, 128), jnp.int32),
        pltpu.VMEM((GS, 128), jnp.int32),
        pltpu.VMEM((128, D), jnp.float32),
        pltpu.VMEM((128, D), jnp.float32),
        pltpu.VMEM_SHARED((TBL, D), jnp.float32),
        pltpu.SemaphoreType.DMA,
        pltpu.SemaphoreType.DMA,
        pltpu.SemaphoreType.DMA,
        pltpu.SemaphoreType.DMA,
    ],
)


def _enc_body(x_ref, we_ref, be_ref, wg_ref, out_ref):
    h = jnp.maximum(
        jnp.dot(x_ref[...], we_ref[...], preferred_element_type=jnp.float32)
        + be_ref[...], 0.0)
    out_ref[...] = jnp.dot(h, wg_ref[...], preferred_element_type=jnp.float32)


_enc = pl.pallas_call(
    _enc_body,
    grid=(G,),
    in_specs=[
        pl.BlockSpec((BM, D), lambda i: (i, 0)),
        pl.BlockSpec((D, D), lambda i: (0, 0)),
        pl.BlockSpec((1, D), lambda i: (0, 0)),
        pl.BlockSpec((D, D), lambda i: (0, 0)),
    ],
    out_specs=pl.BlockSpec((BM, D), lambda i: (i, 0)),
    out_shape=jax.ShapeDtypeStruct((N, D), jnp.float32),
)


def _scale_body(hm_ref, cnt_ref, mp_ref, dis_ref):
    deg = 1.0 + cnt_ref[0, :, 0:1] + cnt_ref[1, :, 0:1]
    dis = lax.rsqrt(deg)
    dis_ref[...] = jnp.broadcast_to(dis, (BM, D))
    mp_ref[...] = hm_ref[...] * dis


_scale = pl.pallas_call(
    _scale_body,
    grid=(G,),
    in_specs=[
        pl.BlockSpec((BM, D), lambda i: (i, 0)),
        pl.BlockSpec((NC, BM, D), lambda i: (0, i, 0)),
    ],
    out_specs=[
        pl.BlockSpec((BM, D), lambda i: (i, 0)),
        pl.BlockSpec((BM, D), lambda i: (i, 0)),
    ],
    out_shape=[
        jax.ShapeDtypeStruct((N, D), jnp.float32),
        jax.ShapeDtypeStruct((N, D), jnp.float32),
    ],
)


def _combine_body(parts_ref, mp_ref, dis_ref, b_ref, w_ref, out_ref):
    dis = dis_ref[...]
    q = parts_ref[0] + parts_ref[1] + mp_ref[...]
    h = jnp.maximum(dis * q + b_ref[...], 0.0)
    out_ref[...] = jnp.dot(
        h, w_ref[...], preferred_element_type=jnp.float32) * dis


_combine = pl.pallas_call(
    _combine_body,
    grid=(G,),
    in_specs=[
        pl.BlockSpec((NC, BM, D), lambda i: (0, i, 0)),
        pl.BlockSpec((BM, D), lambda i: (i, 0)),
        pl.BlockSpec((BM, D), lambda i: (i, 0)),
        pl.BlockSpec((1, D), lambda i: (0, 0)),
        pl.BlockSpec((D, D), lambda i: (0, 0)),
    ],
    out_specs=pl.BlockSpec((BM, D), lambda i: (i, 0)),
    out_shape=jax.ShapeDtypeStruct((N, D), jnp.float32),
)


def _final_body(parts_ref, mp_ref, dis_ref, b_ref, wp1_ref, bp1_ref,
                wp2_ref, bp2_ref, out_ref, acc_ref):
    i = pl.program_id(0)

    @pl.when(i == 0)
    def _():
        acc_ref[...] = jnp.zeros_like(acc_ref)

    h = jnp.maximum(
        dis_ref[...] * (parts_ref[0] + parts_ref[1] + mp_ref[...])
        + b_ref[...], 0.0)
    acc_ref[...] += jnp.sum(h, axis=0, keepdims=True)

    @pl.when(i == G - 1)
    def _():
        g = acc_ref[...] * (1.0 / N)
        z = jnp.maximum(
            jnp.dot(g, wp1_ref[...], preferred_element_type=jnp.float32)
            + bp1_ref[...], 0.0)
        out_ref[...] = jnp.dot(
            z, wp2_ref[...], preferred_element_type=jnp.float32) + bp2_ref[...]


_final = pl.pallas_call(
    _final_body,
    grid=(G,),
    in_specs=[
        pl.BlockSpec((NC, BM, D), lambda i: (0, i, 0)),
        pl.BlockSpec((BM, D), lambda i: (i, 0)),
        pl.BlockSpec((BM, D), lambda i: (i, 0)),
        pl.BlockSpec((1, D), lambda i: (0, 0)),
        pl.BlockSpec((D, D // 2), lambda i: (0, 0)),
        pl.BlockSpec((1, D // 2), lambda i: (0, 0)),
        pl.BlockSpec((D // 2, D), lambda i: (0, 0)),
        pl.BlockSpec((1, D), lambda i: (0, 0)),
    ],
    out_specs=pl.BlockSpec((1, D), lambda i: (0, 0)),
    out_shape=jax.ShapeDtypeStruct((1, D), jnp.float32),
    scratch_shapes=[pltpu.VMEM((1, D), jnp.float32)],
)


def kernel(x, edge_index, W_enc, b_enc, W_g0, b_g0, W_g1, b_g1, W_g2, b_g2,
           W_p1, b_p1, W_p2, b_p2):
    src = edge_index[0]
    dst = edge_index[1]
    pad = EPAD - E
    src3 = jnp.concatenate(
        [src, jnp.zeros((pad,), jnp.int32)]).reshape(TCH, 128)
    pad_dst = DUMMY + jnp.arange(pad, dtype=jnp.int32) % (TBL - N)
    dst3 = jnp.concatenate([dst, pad_dst]).reshape(TCH, 128)
    ones128 = jnp.ones((128, D), jnp.float32)
    zeros128 = jnp.zeros((128, D), jnp.float32)

    counts = _deg(dst3, ones128, zeros128)
    h0m = _enc(x, W_enc, b_enc.reshape(1, D), W_g0)
    mp, disb = _scale(h0m, counts)
    parts = _layer(mp, src3, dst3, zeros128)
    mp = _combine(parts, mp, disb, b_g0.reshape(1, D), W_g1)
    parts = _layer(mp, src3, dst3, zeros128)
    mp = _combine(parts, mp, disb, b_g1.reshape(1, D), W_g2)
    parts = _layer(mp, src3, dst3, zeros128)
    out = _final(parts, mp, disb, b_g2.reshape(1, D), W_p1,
                 b_p1.reshape(1, D // 2), W_p2, b_p2.reshape(1, D))
    return out.reshape(D)


# speedup vs baseline: 1.1165x; 1.0006x over previous
import jax
import jax.numpy as jnp
from jax import lax
from jax.experimental import pallas as pl
from jax.experimental.pallas import tpu as pltpu
from jax.experimental.pallas import tpu_sc as plsc

N = 10000
E = 320000
D = 128

NC = 2
NS = 16
NW = NC * NS
CH = 80
C0 = 128
C1 = 32
GS = 32
TCH = NS * (C0 + C1)
EPAD = TCH * 128
TBL = 10240
STRIPE = TBL // NS
KS = STRIPE // 128
DUMMY = N

BM = 2000
G = N // BM

_mesh = plsc.VectorSubcoreMesh(core_axis_name="c", subcore_axis_name="s")


def _deg_body(dst2, ones128, zeros128, out, dstv, onesv, table, ssem):
    c = lax.axis_index("c")
    s = lax.axis_index("s")
    wid = c * NS + s
    base = s * STRIPE
    for k in range(KS):
        pltpu.sync_copy(zeros128, table.at[pl.ds(base + k * 128, 128)])
    pltpu.sync_copy(dst2.at[pl.ds(wid * CH, CH)], dstv)
    pltpu.sync_copy(ones128, onesv)
    plsc.subcore_barrier()

    def body(t, carry):
        for b in range(4):
            pltpu.async_copy(onesv, table.at[dstv.at[t * 4 + b]], ssem,
                             add=True)
        for b in range(4):
            pltpu.make_async_copy(onesv, table.at[dstv.at[t * 4 + b]],
                                  ssem).wait()
        return carry

    lax.fori_loop(0, CH // 4, body, 0)
    plsc.subcore_barrier()
    for k in range(KS):
        pltpu.sync_copy(table.at[pl.ds(base + k * 128, 128)],
                        out.at[c, pl.ds(base + k * 128, 128)])


_deg = pl.kernel(
    _deg_body,
    mesh=_mesh,
    out_type=jax.ShapeDtypeStruct((NC, TBL, D), jnp.float32),
    scratch_types=[
        pltpu.VMEM((CH, 128), jnp.int32),
        pltpu.VMEM((128, D), jnp.float32),
        pltpu.VMEM_SHARED((TBL, D), jnp.float32),
        pltpu.SemaphoreType.DMA,
    ],
)


def _layer_body(mp, src3, dst3, zeros128, out, srcv, dstv, rowsA, rowsB,
                table, gsemA, gsemB, ssemA, ssemB):
    c = lax.axis_index("c")
    s = lax.axis_index("s")
    wid = c * NS + s
    base = s * STRIPE
    for k in range(KS):
        pltpu.sync_copy(zeros128, table.at[pl.ds(base + k * 128, 128)])
    def fire_gather(j, rows, gsem):
        pltpu.async_copy(mp.at[srcv.at[j]], rows, gsem)

    def drain_gather(j, rows, gsem):
        pltpu.make_async_copy(mp.at[srcv.at[j]], rows, gsem).wait()

    def fire_scatter(j, rows, ssem):
        pltpu.async_copy(rows, table.at[dstv.at[j]], ssem, add=True)

    def drain_scatter(j, rows, ssem):
        pltpu.make_async_copy(rows, table.at[dstv.at[j]], ssem).wait()

    plsc.subcore_barrier()

    cb = lax.select(c == 0, s * C0, NS * C0 + s * C1)
    ng = lax.select(c == 0, C0 // GS, C1 // GS)

    def group(g, carry0):
        off = pl.multiple_of(cb + g * GS, 8)
        pltpu.sync_copy(src3.at[pl.ds(off, GS)], srcv)
        pltpu.sync_copy(dst3.at[pl.ds(off, GS)], dstv)
        fire_gather(0, rowsA, gsemA)

        def body(t, carry):
            jA = 2 * t
            jB = 2 * t + 1
            drain_gather(jA, rowsA, gsemA)
            fire_gather(jB, rowsB, gsemB)
            fire_scatter(jA, rowsA, ssemA)
            drain_scatter(jA, rowsA, ssemA)
            drain_gather(jB, rowsB, gsemB)

            @pl.when(t < GS // 2 - 1)
            def _():
                fire_gather(jA + 2, rowsA, gsemA)

            fire_scatter(jB, rowsB, ssemB)
            drain_scatter(jB, rowsB, ssemB)
            return carry

        lax.fori_loop(0, GS // 2, body, 0)
        return carry0

    lax.fori_loop(0, ng, group, 0)
    plsc.subcore_barrier()
    for k in range(KS):
        pltpu.sync_copy(table.at[pl.ds(base + k * 128, 128)],
                        out.at[c, pl.ds(base + k * 128, 128)])


_layer = pl.kernel(
    _layer_body,
    mesh=_mesh,
    out_type=jax.ShapeDtypeStruct((NC, TBL, D), jnp.float32),
    scratch_types=[
        pltpu.VMEM((GS, 128), jnp.int32),
        pltpu.VMEM((GS, 128), jnp.int32),
        pltpu.VMEM((128, D), jnp.float32),
        pltpu.VMEM((128, D), jnp.float32),
        pltpu.VMEM_SHARED((TBL, D), jnp.float32),
        pltpu.SemaphoreType.DMA,
        pltpu.SemaphoreType.DMA,
        pltpu.SemaphoreType.DMA,
        pltpu.SemaphoreType.DMA,
    ],
)


def _enc_body(x_ref, we_ref, be_ref, wg_ref, out_ref):
    h = jnp.maximum(
        jnp.dot(x_ref[...], we_ref[...], preferred_element_type=jnp.float32)
        + be_ref[...], 0.0)
    out_ref[...] = jnp.dot(h, wg_ref[...], preferred_element_type=jnp.float32)


_enc = pl.pallas_call(
    _enc_body,
    grid=(G,),
    in_specs=[
        pl.BlockSpec((BM, D), lambda i: (i, 0)),
        pl.BlockSpec((D, D), lambda i: (0, 0)),
        pl.BlockSpec((1, D), lambda i: (0, 0)),
        pl.BlockSpec((D, D), lambda i: (0, 0)),
    ],
    out_specs=pl.BlockSpec((BM, D), lambda i: (i, 0)),
    out_shape=jax.ShapeDtypeStruct((N, D), jnp.float32),
)


def _scale_body(hm_ref, cnt_ref, mp_ref, dis_ref):
    deg = 1.0 + cnt_ref[0, :, 0:1] + cnt_ref[1, :, 0:1]
    dis = lax.rsqrt(deg)
    dis_ref[...] = jnp.broadcast_to(dis, (BM, D))
    mp_ref[...] = hm_ref[...] * dis


_scale = pl.pallas_call(
    _scale_body,
    grid=(G,),
    in_specs=[
        pl.BlockSpec((BM, D), lambda i: (i, 0)),
        pl.BlockSpec((NC, BM, D), lambda i: (0, i, 0)),
    ],
    out_specs=[
        pl.BlockSpec((BM, D), lambda i: (i, 0)),
        pl.BlockSpec((BM, D), lambda i: (i, 0)),
    ],
    out_shape=[
        jax.ShapeDtypeStruct((N, D), jnp.float32),
        jax.ShapeDtypeStruct((N, D), jnp.float32),
    ],
)


def _combine_body(parts_ref, mp_ref, dis_ref, b_ref, w_ref, out_ref):
    dis = dis_ref[...]
    q = parts_ref[0] + parts_ref[1] + mp_ref[...]
    h = jnp.maximum(dis * q + b_ref[...], 0.0)
    out_ref[...] = jnp.dot(
        h, w_ref[...], preferred_element_type=jnp.float32) * dis


_combine = pl.pallas_call(
    _combine_body,
    grid=(G,),
    in_specs=[
        pl.BlockSpec((NC, BM, D), lambda i: (0, i, 0)),
        pl.BlockSpec((BM, D), lambda i: (i, 0)),
        pl.BlockSpec((BM, D), lambda i: (i, 0)),
        pl.BlockSpec((1, D), lambda i: (0, 0)),
        pl.BlockSpec((D, D), lambda i: (0, 0)),
    ],
    out_specs=pl.BlockSpec((BM, D), lambda i: (i, 0)),
    out_shape=jax.ShapeDtypeStruct((N, D), jnp.float32),
)


def _final_body(parts_ref, mp_ref, dis_ref, b_ref, wp1_ref, bp1_ref,
                wp2_ref, bp2_ref, out_ref, acc_ref):
    i = pl.program_id(0)

    @pl.when(i == 0)
    def _():
        acc_ref[...] = jnp.zeros_like(acc_ref)

    h = jnp.maximum(
        dis_ref[...] * (parts_ref[0] + parts_ref[1] + mp_ref[...])
        + b_ref[...], 0.0)
    acc_ref[...] += jnp.sum(h, axis=0, keepdims=True)

    @pl.when(i == G - 1)
    def _():
        g = acc_ref[...] * (1.0 / N)
        z = jnp.maximum(
            jnp.dot(g, wp1_ref[...], preferred_element_type=jnp.float32)
            + bp1_ref[...], 0.0)
        out_ref[...] = jnp.dot(
            z, wp2_ref[...], preferred_element_type=jnp.float32) + bp2_ref[...]


_final = pl.pallas_call(
    _final_body,
    grid=(G,),
    in_specs=[
        pl.BlockSpec((NC, BM, D), lambda i: (0, i, 0)),
        pl.BlockSpec((BM, D), lambda i: (i, 0)),
        pl.BlockSpec((BM, D), lambda i: (i, 0)),
        pl.BlockSpec((1, D), lambda i: (0, 0)),
        pl.BlockSpec((D, D // 2), lambda i: (0, 0)),
        pl.BlockSpec((1, D // 2), lambda i: (0, 0)),
        pl.BlockSpec((D // 2, D), lambda i: (0, 0)),
        pl.BlockSpec((1, D), lambda i: (0, 0)),
    ],
    out_specs=pl.BlockSpec((1, D), lambda i: (0, 0)),
    out_shape=jax.ShapeDtypeStruct((1, D), jnp.float32),
    scratch_shapes=[pltpu.VMEM((1, D), jnp.float32)],
)


def kernel(x, edge_index, W_enc, b_enc, W_g0, b_g0, W_g1, b_g1, W_g2, b_g2,
           W_p1, b_p1, W_p2, b_p2):
    src = edge_index[0]
    dst = edge_index[1]
    pad = EPAD - E
    src3 = jnp.concatenate(
        [src, jnp.zeros((pad,), jnp.int32)]).reshape(TCH, 128)
    pad_dst = DUMMY + jnp.arange(pad, dtype=jnp.int32) % (TBL - N)
    dst3 = jnp.concatenate([dst, pad_dst]).reshape(TCH, 128)
    ones128 = jnp.ones((128, D), jnp.float32)
    zeros128 = jnp.zeros((128, D), jnp.float32)

    counts = _deg(dst3, ones128, zeros128)
    h0m = _enc(x, W_enc, b_enc.reshape(1, D), W_g0)
    mp, disb = _scale(h0m, counts)
    parts = _layer(mp, src3, dst3, zeros128)
    mp = _combine(parts, mp, disb, b_g0.reshape(1, D), W_g1)
    parts = _layer(mp, src3, dst3, zeros128)
    mp = _combine(parts, mp, disb, b_g1.reshape(1, D), W_g2)
    parts = _layer(mp, src3, dst3, zeros128)
    out = _final(parts, mp, disb, b_g2.reshape(1, D), W_p1,
                 b_p1.reshape(1, D // 2), W_p2, b_p2.reshape(1, D))
    return out.reshape(D)


# speedup vs baseline: 1.2553x; 1.1244x over previous
import jax
import jax.numpy as jnp
from jax import lax
from jax.experimental import pallas as pl
from jax.experimental.pallas import tpu as pltpu
from jax.experimental.pallas import tpu_sc as plsc

N = 10000
E = 320000
D = 128

NC = 2
NS = 16
NW = NC * NS
CH = 80
C0 = 144
C1 = 16
GS = 16
TCH = NS * (C0 + C1)
EPAD = TCH * 128
TBL = 10240
STRIPE = TBL // NS
KS = STRIPE // 128
DUMMY = N

BM = 2000
G = N // BM

_mesh = plsc.VectorSubcoreMesh(core_axis_name="c", subcore_axis_name="s")


def _deg_body(dst2, ones128, zeros128, out, dstv, onesv, table, ssem):
    c = lax.axis_index("c")
    s = lax.axis_index("s")
    wid = c * NS + s
    base = s * STRIPE
    for k in range(KS):
        pltpu.sync_copy(zeros128, table.at[pl.ds(base + k * 128, 128)])
    pltpu.sync_copy(dst2.at[pl.ds(wid * CH, CH)], dstv)
    pltpu.sync_copy(ones128, onesv)
    plsc.subcore_barrier()

    def body(t, carry):
        for b in range(4):
            pltpu.async_copy(onesv, table.at[dstv.at[t * 4 + b]], ssem,
                             add=True)
        for b in range(4):
            pltpu.make_async_copy(onesv, table.at[dstv.at[t * 4 + b]],
                                  ssem).wait()
        return carry

    lax.fori_loop(0, CH // 4, body, 0)
    plsc.subcore_barrier()
    for k in range(KS):
        pltpu.sync_copy(table.at[pl.ds(base + k * 128, 128)],
                        out.at[c, pl.ds(base + k * 128, 128)])


_deg = pl.kernel(
    _deg_body,
    mesh=_mesh,
    out_type=jax.ShapeDtypeStruct((NC, TBL, D), jnp.float32),
    scratch_types=[
        pltpu.VMEM((CH, 128), jnp.int32),
        pltpu.VMEM((128, D), jnp.float32),
        pltpu.VMEM_SHARED((TBL, D), jnp.float32),
        pltpu.SemaphoreType.DMA,
    ],
)


def _layer_body(mp, src3, dst3, zeros128, out, srcv, dstv, rowsA, rowsB,
                table, gsemA, gsemB, ssemA, ssemB):
    c = lax.axis_index("c")
    s = lax.axis_index("s")
    wid = c * NS + s
    base = s * STRIPE
    for k in range(KS):
        pltpu.sync_copy(zeros128, table.at[pl.ds(base + k * 128, 128)])
    def fire_gather(j, rows, gsem):
        pltpu.async_copy(mp.at[srcv.at[j]], rows, gsem)

    def drain_gather(j, rows, gsem):
        pltpu.make_async_copy(mp.at[srcv.at[j]], rows, gsem).wait()

    def fire_scatter(j, rows, ssem):
        pltpu.async_copy(rows, table.at[dstv.at[j]], ssem, add=True)

    def drain_scatter(j, rows, ssem):
        pltpu.make_async_copy(rows, table.at[dstv.at[j]], ssem).wait()

    plsc.subcore_barrier()

    cb = lax.select(c == 0, s * C0, NS * C0 + s * C1)
    ng = lax.select(c == 0, C0 // GS, C1 // GS)

    def group(g, carry0):
        off = pl.multiple_of(cb + g * GS, 8)
        pltpu.sync_copy(src3.at[pl.ds(off, GS)], srcv)
        pltpu.sync_copy(dst3.at[pl.ds(off, GS)], dstv)
        fire_gather(0, rowsA, gsemA)

        def body(t, carry):
            jA = 2 * t
            jB = 2 * t + 1
            drain_gather(jA, rowsA, gsemA)
            fire_gather(jB, rowsB, gsemB)
            fire_scatter(jA, rowsA, ssemA)
            drain_scatter(jA, rowsA, ssemA)
            drain_gather(jB, rowsB, gsemB)

            @pl.when(t < GS // 2 - 1)
            def _():
                fire_gather(jA + 2, rowsA, gsemA)

            fire_scatter(jB, rowsB, ssemB)
            drain_scatter(jB, rowsB, ssemB)
            return carry

        lax.fori_loop(0, GS // 2, body, 0)
        return carry0

    lax.fori_loop(0, ng, group, 0)
    plsc.subcore_barrier()
    for k in range(KS):
        pltpu.sync_copy(table.at[pl.ds(base + k * 128, 128)],
                        out.at[c, pl.ds(base + k * 128, 128)])


_layer = pl.kernel(
    _layer_body,
    mesh=_mesh,
    out_type=jax.ShapeDtypeStruct((NC, TBL, D), jnp.float32),
    scratch_types=[
        pltpu.VMEM((GS, 128), jnp.int32),
        pltpu.VMEM((GS, 128), jnp.int32),
        pltpu.VMEM((128, D), jnp.float32),
        pltpu.VMEM((128, D), jnp.float32),
        pltpu.VMEM_SHARED((TBL, D), jnp.float32),
        pltpu.SemaphoreType.DMA,
        pltpu.SemaphoreType.DMA,
        pltpu.SemaphoreType.DMA,
        pltpu.SemaphoreType.DMA,
    ],
)


def _enc_body(x_ref, we_ref, be_ref, wg_ref, out_ref):
    h = jnp.maximum(
        jnp.dot(x_ref[...], we_ref[...], preferred_element_type=jnp.float32)
        + be_ref[...], 0.0)
    out_ref[...] = jnp.dot(h, wg_ref[...], preferred_element_type=jnp.float32)


_enc = pl.pallas_call(
    _enc_body,
    grid=(G,),
    in_specs=[
        pl.BlockSpec((BM, D), lambda i: (i, 0)),
        pl.BlockSpec((D, D), lambda i: (0, 0)),
        pl.BlockSpec((1, D), lambda i: (0, 0)),
        pl.BlockSpec((D, D), lambda i: (0, 0)),
    ],
    out_specs=pl.BlockSpec((BM, D), lambda i: (i, 0)),
    out_shape=jax.ShapeDtypeStruct((N, D), jnp.float32),
)


def _scale_body(hm_ref, cnt_ref, mp_ref, dis_ref):
    deg = 1.0 + cnt_ref[0, :, 0:1] + cnt_ref[1, :, 0:1]
    dis = lax.rsqrt(deg)
    dis_ref[...] = jnp.broadcast_to(dis, (BM, D))
    mp_ref[...] = hm_ref[...] * dis


_scale = pl.pallas_call(
    _scale_body,
    grid=(G,),
    in_specs=[
        pl.BlockSpec((BM, D), lambda i: (i, 0)),
        pl.BlockSpec((NC, BM, D), lambda i: (0, i, 0)),
    ],
    out_specs=[
        pl.BlockSpec((BM, D), lambda i: (i, 0)),
        pl.BlockSpec((BM, D), lambda i: (i, 0)),
    ],
    out_shape=[
        jax.ShapeDtypeStruct((N, D), jnp.float32),
        jax.ShapeDtypeStruct((N, D), jnp.float32),
    ],
)


def _combine_body(parts_ref, mp_ref, dis_ref, b_ref, w_ref, out_ref):
    dis = dis_ref[...]
    q = parts_ref[0] + parts_ref[1] + mp_ref[...]
    h = jnp.maximum(dis * q + b_ref[...], 0.0)
    out_ref[...] = jnp.dot(
        h, w_ref[...], preferred_element_type=jnp.float32) * dis


_combine = pl.pallas_call(
    _combine_body,
    grid=(G,),
    in_specs=[
        pl.BlockSpec((NC, BM, D), lambda i: (0, i, 0)),
        pl.BlockSpec((BM, D), lambda i: (i, 0)),
        pl.BlockSpec((BM, D), lambda i: (i, 0)),
        pl.BlockSpec((1, D), lambda i: (0, 0)),
        pl.BlockSpec((D, D), lambda i: (0, 0)),
    ],
    out_specs=pl.BlockSpec((BM, D), lambda i: (i, 0)),
    out_shape=jax.ShapeDtypeStruct((N, D), jnp.float32),
)


def _final_body(parts_ref, mp_ref, dis_ref, b_ref, wp1_ref, bp1_ref,
                wp2_ref, bp2_ref, out_ref, acc_ref):
    i = pl.program_id(0)

    @pl.when(i == 0)
    def _():
        acc_ref[...] = jnp.zeros_like(acc_ref)

    h = jnp.maximum(
        dis_ref[...] * (parts_ref[0] + parts_ref[1] + mp_ref[...])
        + b_ref[...], 0.0)
    acc_ref[...] += jnp.sum(h, axis=0, keepdims=True)

    @pl.when(i == G - 1)
    def _():
        g = acc_ref[...] * (1.0 / N)
        z = jnp.maximum(
            jnp.dot(g, wp1_ref[...], preferred_element_type=jnp.float32)
            + bp1_ref[...], 0.0)
        out_ref[...] = jnp.dot(
            z, wp2_ref[...], preferred_element_type=jnp.float32) + bp2_ref[...]


_final = pl.pallas_call(
    _final_body,
    grid=(G,),
    in_specs=[
        pl.BlockSpec((NC, BM, D), lambda i: (0, i, 0)),
        pl.BlockSpec((BM, D), lambda i: (i, 0)),
        pl.BlockSpec((BM, D), lambda i: (i, 0)),
        pl.BlockSpec((1, D), lambda i: (0, 0)),
        pl.BlockSpec((D, D // 2), lambda i: (0, 0)),
        pl.BlockSpec((1, D // 2), lambda i: (0, 0)),
        pl.BlockSpec((D // 2, D), lambda i: (0, 0)),
        pl.BlockSpec((1, D), lambda i: (0, 0)),
    ],
    out_specs=pl.BlockSpec((1, D), lambda i: (0, 0)),
    out_shape=jax.ShapeDtypeStruct((1, D), jnp.float32),
    scratch_shapes=[pltpu.VMEM((1, D), jnp.float32)],
)


def kernel(x, edge_index, W_enc, b_enc, W_g0, b_g0, W_g1, b_g1, W_g2, b_g2,
           W_p1, b_p1, W_p2, b_p2):
    src = edge_index[0]
    dst = edge_index[1]
    pad = EPAD - E
    src3 = jnp.concatenate(
        [src, jnp.zeros((pad,), jnp.int32)]).reshape(TCH, 128)
    pad_dst = DUMMY + jnp.arange(pad, dtype=jnp.int32) % (TBL - N)
    dst3 = jnp.concatenate([dst, pad_dst]).reshape(TCH, 128)
    ones128 = jnp.ones((128, D), jnp.float32)
    zeros128 = jnp.zeros((128, D), jnp.float32)

    counts = _deg(dst3, ones128, zeros128)
    h0m = _enc(x, W_enc, b_enc.reshape(1, D), W_g0)
    mp, disb = _scale(h0m, counts)
    parts = _layer(mp, src3, dst3, zeros128)
    mp = _combine(parts, mp, disb, b_g0.reshape(1, D), W_g1)
    parts = _layer(mp, src3, dst3, zeros128)
    mp = _combine(parts, mp, disb, b_g1.reshape(1, D), W_g2)
    parts = _layer(mp, src3, dst3, zeros128)
    out = _final(parts, mp, disb, b_g2.reshape(1, D), W_p1,
                 b_p1.reshape(1, D // 2), W_p2, b_p2.reshape(1, D))
    return out.reshape(D)
